# baseline (speedup 1.0000x reference)
import jax
import jax.numpy as jnp
from jax import lax
from jax.experimental import pallas as pl
from jax.experimental.pallas import tpu as pltpu
from jax.experimental.pallas import tpu_sc as plsc

NUM_EMBEDDINGS = 1000000
EMBEDDING_DIM = 64
BATCH = 16384
SEQ_LEN = 50

_INFO = plsc.get_sparse_core_info()
NC, NS = _INFO.num_cores, _INFO.num_subcores
NW = NC * NS

LANES = 128
N_BT = BATCH // LANES
BT_PER_W = N_BT // NW
GROUPS = SEQ_LEN * BT_PER_W
NBUF = 4


def _body(idx_hbm, table_hbm, out_hbm, idx_v, rows_v, tr_v, *sems):
    wid = lax.axis_index("s") * NC + lax.axis_index("c")
    sem_g = sems[:NBUF]
    sem_s = sems[NBUF:]

    pltpu.sync_copy(idx_hbm.at[wid], idx_v)

    iota = lax.iota(jnp.int32, 16)

    def issue_gather(g, b):
        pltpu.async_copy(table_hbm.at[idx_v.at[g]], rows_v.at[b], sem_g[b])

    def wait_gather(b):
        pltpu.make_async_copy(
            table_hbm.at[pl.ds(0, LANES)], rows_v.at[b], sem_g[b]
        ).wait()

    def transpose(b):
        def dt_loop(dt, carry):
            for dl in range(8):
                col = jnp.broadcast_to(dt * 8 + dl, (16,)).astype(jnp.int32)
                for c in range(8):
                    v = plsc.load_gather(rows_v.at[b], [c * 16 + iota, col])
                    tr_v[b, dt, dl, pl.ds(c * 16, 16)] = v
            return carry

        lax.fori_loop(0, 8, dt_loop, 0)

    def issue_store(g, b):
        s = g // BT_PER_W
        bt = wid * BT_PER_W + lax.rem(g, BT_PER_W)
        pltpu.async_copy(tr_v.at[b], out_hbm.at[s, :, bt], sem_s[b])

    def wait_store(b):
        pltpu.make_async_copy(
            tr_v.at[b], out_hbm.at[0, :, 0], sem_s[b]
        ).wait()

    for b in range(NBUF):
        issue_gather(b, b)

    def outer(o, carry):
        for b in range(NBUF):
            g = o * NBUF + b
            wait_gather(b)

            @pl.when(g >= NBUF)
            def _():
                wait_store(b)

            transpose(b)
            issue_store(g, b)

            @pl.when(g + NBUF < GROUPS)
            def _():
                issue_gather(g + NBUF, b)

        return carry

    lax.fori_loop(0, GROUPS // NBUF, outer, 0)
    for b in range(NBUF):
        wait_store(b)


@jax.jit
def _gather(idx_grouped, table):
    mesh = plsc.VectorSubcoreMesh(core_axis_name="c", subcore_axis_name="s")
    run = pl.kernel(
        _body,
        out_type=jax.ShapeDtypeStruct(
            (SEQ_LEN, EMBEDDING_DIM // 8, N_BT, 8, LANES), jnp.float32
        ),
        mesh=mesh,
        scratch_types=[
            pltpu.VMEM((GROUPS, LANES), jnp.int32),
            pltpu.VMEM((NBUF, LANES, EMBEDDING_DIM), jnp.float32),
            pltpu.VMEM((NBUF, EMBEDDING_DIM // 8, 8, LANES), jnp.float32),
        ] + [pltpu.SemaphoreType.DMA] * (2 * NBUF),
        compiler_params=pltpu.CompilerParams(
            use_tc_tiling_on_sc=False, needs_layout_passes=False
        ),
    )
    return run(idx_grouped, table)


def kernel(token_ids, table):
    tt = token_ids.astype(jnp.int32).T
    idx = (
        tt.reshape(SEQ_LEN, NW, BT_PER_W, LANES)
        .transpose(1, 0, 2, 3)
        .reshape(NW, GROUPS, LANES)
    )
    out5 = _gather(idx, table)
    return out5.transpose(2, 4, 0, 1, 3).reshape(BATCH, SEQ_LEN, EMBEDDING_DIM)


# speedup vs baseline: 1.4827x; 1.4827x over previous
import jax
import jax.numpy as jnp
from jax import lax
from jax.experimental import pallas as pl
from jax.experimental.pallas import tpu as pltpu
from jax.experimental.pallas import tpu_sc as plsc

NUM_EMBEDDINGS = 1000000
EMBEDDING_DIM = 64
BATCH = 16384
SEQ_LEN = 50

_INFO = plsc.get_sparse_core_info()
NC, NS = _INFO.num_cores, _INFO.num_subcores
NW = NC * NS

LANES = 128
N_BT = BATCH // LANES
BT_PER_W = N_BT // NW
GROUPS = SEQ_LEN * BT_PER_W
NBUF = 4


def _body(idx_hbm, table_hbm, out_hbm, idx_v, rows_v, tr_v, *sems):
    wid = lax.axis_index("s") * NC + lax.axis_index("c")
    sem_g = sems[:NBUF]
    sem_s = sems[NBUF:]

    pltpu.sync_copy(idx_hbm.at[wid], idx_v)

    iota = lax.iota(jnp.int32, 16)

    def issue_gather(g, b):
        pltpu.async_copy(table_hbm.at[idx_v.at[g]], rows_v.at[b], sem_g[b])

    def wait_gather(b):
        pltpu.make_async_copy(
            table_hbm.at[pl.ds(0, LANES)], rows_v.at[b], sem_g[b]
        ).wait()

    def transpose(b):
        @plsc.parallel_loop(0, EMBEDDING_DIM, unroll=4)
        def _(d):
            dt = d // 8
            dl = lax.rem(d, 8)
            col = jnp.broadcast_to(d, (16,)).astype(jnp.int32)
            for c in range(8):
                v = plsc.load_gather(rows_v.at[b], [c * 16 + iota, col])
                tr_v[b, dt, dl, pl.ds(c * 16, 16)] = v

    def issue_store(g, b):
        s = g // BT_PER_W
        bt = wid * BT_PER_W + lax.rem(g, BT_PER_W)
        pltpu.async_copy(tr_v.at[b], out_hbm.at[s, :, bt], sem_s[b])

    def wait_store(b):
        pltpu.make_async_copy(
            tr_v.at[b], out_hbm.at[0, :, 0], sem_s[b]
        ).wait()

    for b in range(NBUF):
        issue_gather(b, b)

    def outer(o, carry):
        for b in range(NBUF):
            g = o * NBUF + b
            wait_gather(b)

            @pl.when(g >= NBUF)
            def _():
                wait_store(b)

            transpose(b)
            issue_store(g, b)

            @pl.when(g + NBUF < GROUPS)
            def _():
                issue_gather(g + NBUF, b)

        return carry

    lax.fori_loop(0, GROUPS // NBUF, outer, 0)
    for b in range(NBUF):
        wait_store(b)


@jax.jit
def _gather(idx_grouped, table):
    mesh = plsc.VectorSubcoreMesh(core_axis_name="c", subcore_axis_name="s")
    run = pl.kernel(
        _body,
        out_type=jax.ShapeDtypeStruct(
            (SEQ_LEN, EMBEDDING_DIM // 8, N_BT, 8, LANES), jnp.float32
        ),
        mesh=mesh,
        scratch_types=[
            pltpu.VMEM((GROUPS, LANES), jnp.int32),
            pltpu.VMEM((NBUF, LANES, EMBEDDING_DIM), jnp.float32),
            pltpu.VMEM((NBUF, EMBEDDING_DIM // 8, 8, LANES), jnp.float32),
        ] + [pltpu.SemaphoreType.DMA] * (2 * NBUF),
        compiler_params=pltpu.CompilerParams(
            use_tc_tiling_on_sc=False, needs_layout_passes=False
        ),
    )
    return run(idx_grouped, table)


def kernel(token_ids, table):
    tt = token_ids.astype(jnp.int32).T
    idx = (
        tt.reshape(SEQ_LEN, NW, BT_PER_W, LANES)
        .transpose(1, 0, 2, 3)
        .reshape(NW, GROUPS, LANES)
    )
    out5 = _gather(idx, table)
    return out5.transpose(2, 4, 0, 1, 3).reshape(BATCH, SEQ_LEN, EMBEDDING_DIM)


# speedup vs baseline: 2.4797x; 1.6724x over previous
import jax
import jax.numpy as jnp
from jax import lax
from jax.experimental import pallas as pl
from jax.experimental.pallas import tpu as pltpu
from jax.experimental.pallas import tpu_sc as plsc

NUM_EMBEDDINGS = 1000000
EMBEDDING_DIM = 64
BATCH = 16384
SEQ_LEN = 50

_INFO = plsc.get_sparse_core_info()
NC, NS = _INFO.num_cores, _INFO.num_subcores
NW = NC * NS

LANES = 128
N_BT = BATCH // LANES
BT_PER_W = N_BT // NW
GROUPS = SEQ_LEN * BT_PER_W
NBUF = 4


def _body(idx_hbm, table_hbm, out_hbm, idx_v, rows_v, tr_v, *sems):
    wid = lax.axis_index("s") * NC + lax.axis_index("c")
    sem_g = sems[:NBUF]
    sem_s = sems[NBUF:]

    pltpu.sync_copy(idx_hbm.at[wid], idx_v)

    iota = lax.iota(jnp.int32, 16)

    def issue_gather(g, b):
        pltpu.async_copy(table_hbm.at[idx_v.at[g]], rows_v.at[b], sem_g[b])

    def wait_gather(b):
        pltpu.make_async_copy(
            table_hbm.at[pl.ds(0, LANES)], rows_v.at[b], sem_g[b]
        ).wait()

    dt_idx = [(c * 16 + iota) // 8 for c in range(EMBEDDING_DIM // 16)]
    dl_idx = [lax.rem(c * 16 + iota, 8) for c in range(EMBEDDING_DIM // 16)]

    def transpose(b):
        @plsc.parallel_loop(0, LANES, unroll=4)
        def _(bl):
            blv = jnp.broadcast_to(bl, (16,)).astype(jnp.int32)
            for c in range(EMBEDDING_DIM // 16):
                v = rows_v[b, bl, pl.ds(c * 16, 16)]
                plsc.store_scatter(tr_v.at[b], [dt_idx[c], dl_idx[c], blv], v)

    def issue_store(g, b):
        s = g // BT_PER_W
        bt = wid * BT_PER_W + lax.rem(g, BT_PER_W)
        pltpu.async_copy(
            tr_v.at[b, :, :, pl.ds(0, LANES)], out_hbm.at[s, :, bt], sem_s[b]
        )

    def wait_store(b):
        pltpu.make_async_copy(
            tr_v.at[b, :, :, pl.ds(0, LANES)], out_hbm.at[0, :, 0], sem_s[b]
        ).wait()

    for b in range(NBUF):
        issue_gather(b, b)

    def outer(o, carry):
        for b in range(NBUF):
            g = o * NBUF + b
            wait_gather(b)

            @pl.when(g >= NBUF)
            def _():
                wait_store(b)

            transpose(b)
            issue_store(g, b)

            @pl.when(g + NBUF < GROUPS)
            def _():
                issue_gather(g + NBUF, b)

        return carry

    lax.fori_loop(0, GROUPS // NBUF, outer, 0)
    for b in range(NBUF):
        wait_store(b)


@jax.jit
def _gather(idx_grouped, table):
    mesh = plsc.VectorSubcoreMesh(core_axis_name="c", subcore_axis_name="s")
    run = pl.kernel(
        _body,
        out_type=jax.ShapeDtypeStruct(
            (SEQ_LEN, EMBEDDING_DIM // 8, N_BT, 8, LANES), jnp.float32
        ),
        mesh=mesh,
        scratch_types=[
            pltpu.VMEM((GROUPS, LANES), jnp.int32),
            pltpu.VMEM((NBUF, LANES, EMBEDDING_DIM), jnp.float32),
            pltpu.VMEM((NBUF, EMBEDDING_DIM // 8, 8, LANES + 1), jnp.float32),
        ] + [pltpu.SemaphoreType.DMA] * (2 * NBUF),
        compiler_params=pltpu.CompilerParams(
            use_tc_tiling_on_sc=False, needs_layout_passes=False
        ),
    )
    return run(idx_grouped, table)


def kernel(token_ids, table):
    tt = token_ids.astype(jnp.int32).T
    idx = (
        tt.reshape(SEQ_LEN, NW, BT_PER_W, LANES)
        .transpose(1, 0, 2, 3)
        .reshape(NW, GROUPS, LANES)
    )
    out5 = _gather(idx, table)
    return out5.transpose(2, 4, 0, 1, 3).reshape(BATCH, SEQ_LEN, EMBEDDING_DIM)


# speedup vs baseline: 2.7123x; 1.0938x over previous
import jax
import jax.numpy as jnp
from jax import lax
from jax.experimental import pallas as pl
from jax.experimental.pallas import tpu as pltpu
from jax.experimental.pallas import tpu_sc as plsc

NUM_EMBEDDINGS = 1000000
EMBEDDING_DIM = 64
BATCH = 16384
SEQ_LEN = 50

_INFO = plsc.get_sparse_core_info()
NC, NS = _INFO.num_cores, _INFO.num_subcores
NW = NC * NS

LANES = 128
N_BT = BATCH // LANES
BT_PER_W = N_BT // NW
GROUPS = SEQ_LEN * BT_PER_W
NBUF = 4


def _body(idx_hbm, table_hbm, out_hbm, idx_v, rows_v, tr_v, *sems):
    wid = lax.axis_index("s") * NC + lax.axis_index("c")
    sem_g = sems[:NBUF]
    sem_s = sems[NBUF:]

    pltpu.sync_copy(idx_hbm.at[wid], idx_v)

    iota = lax.iota(jnp.int32, 16)

    def issue_gather(g, b):
        pltpu.async_copy(table_hbm.at[idx_v.at[g]], rows_v.at[b], sem_g[b])

    def wait_gather(b):
        pltpu.make_async_copy(
            table_hbm.at[pl.ds(0, LANES)], rows_v.at[b], sem_g[b]
        ).wait()

    dt_idx = [(c * 16 + iota) // 8 for c in range(EMBEDDING_DIM // 16)]
    dl_idx = [lax.rem(c * 16 + iota, 8) for c in range(EMBEDDING_DIM // 16)]

    def transpose(b):
        @plsc.parallel_loop(0, LANES, unroll=4)
        def _(bl):
            blv = jnp.broadcast_to(bl, (16,)).astype(jnp.int32)
            for c in range(EMBEDDING_DIM // 16):
                v = rows_v[b, bl, pl.ds(c * 16, 16)]
                plsc.store_scatter(tr_v.at[b], [dt_idx[c], dl_idx[c], blv], v)

    def issue_store(g, b):
        s = g // BT_PER_W
        bt = wid * BT_PER_W + lax.rem(g, BT_PER_W)
        pltpu.async_copy(
            tr_v.at[b, :, :, pl.ds(0, LANES)], out_hbm.at[s, :, bt], sem_s[b]
        )

    def wait_store(b):
        pltpu.make_async_copy(
            tr_v.at[b, :, :, pl.ds(0, LANES)], out_hbm.at[0, :, 0], sem_s[b]
        ).wait()

    for b in range(NBUF):
        issue_gather(b, b)

    def outer(o, carry):
        for b in range(NBUF):
            g = o * NBUF + b
            wait_gather(b)

            @pl.when(g >= NBUF)
            def _():
                wait_store(b)

            transpose(b)
            issue_store(g, b)

            @pl.when(g + NBUF < GROUPS)
            def _():
                issue_gather(g + NBUF, b)

        return carry

    lax.fori_loop(0, GROUPS // NBUF, outer, 0)
    for b in range(NBUF):
        wait_store(b)


@jax.jit
def _gather(idx_grouped, table):
    mesh = plsc.VectorSubcoreMesh(core_axis_name="c", subcore_axis_name="s")
    run = pl.kernel(
        _body,
        out_type=jax.ShapeDtypeStruct(
            (SEQ_LEN, EMBEDDING_DIM // 8, N_BT, 8, LANES), jnp.float32
        ),
        mesh=mesh,
        scratch_types=[
            pltpu.VMEM((GROUPS, LANES), jnp.int32),
            pltpu.VMEM((NBUF, LANES, EMBEDDING_DIM), jnp.float32),
            pltpu.VMEM((NBUF, EMBEDDING_DIM // 8, 8, LANES + 1), jnp.float32),
        ] + [pltpu.SemaphoreType.DMA] * (2 * NBUF),
        compiler_params=pltpu.CompilerParams(
            use_tc_tiling_on_sc=False, needs_layout_passes=False
        ),
    )
    return run(idx_grouped, table)


def kernel(token_ids, table):
    tt = token_ids.astype(jnp.int32).T
    idx = (
        tt.reshape(SEQ_LEN, NW, BT_PER_W, LANES)
        .transpose(1, 0, 2, 3)
        .reshape(NW, GROUPS, LANES)
    )
    tpad = jnp.pad(table, ((0, 0), (0, EMBEDDING_DIM)))
    t2 = tpad.reshape(2 * NUM_EMBEDDINGS, EMBEDDING_DIM)
    out5 = _gather(idx * 2, t2)
    return out5.transpose(2, 4, 0, 1, 3).reshape(BATCH, SEQ_LEN, EMBEDDING_DIM)


# speedup vs baseline: 3.3580x; 1.2381x over previous
import jax
import jax.numpy as jnp
from jax import lax
from jax.experimental import pallas as pl
from jax.experimental.pallas import tpu as pltpu
from jax.experimental.pallas import tpu_sc as plsc

NUM_EMBEDDINGS = 1000000
EMBEDDING_DIM = 64
BATCH = 16384
SEQ_LEN = 50

_INFO = plsc.get_sparse_core_info()
NC, NS = _INFO.num_cores, _INFO.num_subcores
NW = NC * NS

LANES = 128
N_BT = BATCH // LANES
BT_PER_W = N_BT // NW
GROUPS = SEQ_LEN * BT_PER_W
NBUF = 4

NPAD = 1000064
NT = NPAD // LANES
K1_STEPS = -(-NT // NW)
K1_NBUF = 4
SOUT_STRIDE = 69


def _tbody(t4_hbm, out1_hbm, sin_v, sout_v, *sems):
    wid = lax.axis_index("s") * NC + lax.axis_index("c")
    sem_i = sems[:K1_NBUF]
    sem_o = sems[K1_NBUF:]
    iota = lax.iota(jnp.int32, 16)

    def issue_load(k, b):
        pltpu.async_copy(t4_hbm.at[:, k * NW + wid], sin_v.at[b], sem_i[b])

    def wait_load(b):
        pltpu.make_async_copy(t4_hbm.at[:, 0], sin_v.at[b], sem_i[b]).wait()

    def transpose(b):
        @plsc.parallel_loop(0, EMBEDDING_DIM, unroll=4)
        def _(d):
            dt = d // 8
            dl = lax.rem(d, 8)
            dcol = jnp.broadcast_to(d, (16,)).astype(jnp.int32)
            for c in range(8):
                v = sin_v[b, dt, dl, pl.ds(c * 16, 16)]
                plsc.store_scatter(sout_v.at[b], [c * 16 + iota, dcol], v)

    def issue_store(k, b):
        it = k * NW + wid
        pltpu.async_copy(
            sout_v.at[b, :, pl.ds(0, EMBEDDING_DIM)],
            out1_hbm.at[pl.ds(it * LANES, LANES), 0],
            sem_o[b],
        )

    def wait_store(b):
        pltpu.make_async_copy(
            sout_v.at[b, :, pl.ds(0, EMBEDDING_DIM)],
            out1_hbm.at[pl.ds(0, LANES), 0],
            sem_o[b],
        ).wait()

    def live(k):
        return k * NW + wid < NT

    for b in range(K1_NBUF):

        @pl.when(live(b))
        def _():
            issue_load(b, b)

    def outer(o, carry):
        for b in range(K1_NBUF):
            k = o * K1_NBUF + b

            @pl.when(live(k))
            def _():
                wait_load(b)

                @pl.when(k >= K1_NBUF)
                def _():
                    wait_store(b)

                transpose(b)
                issue_store(k, b)

            @pl.when(live(k + K1_NBUF))
            def _():
                issue_load(k + K1_NBUF, b)

        return carry

    lax.fori_loop(0, -(-K1_STEPS // K1_NBUF), outer, 0)
    for b in range(K1_NBUF):
        wait_store(b)


@jax.jit
def _transpose_table(t4):
    mesh = plsc.VectorSubcoreMesh(core_axis_name="c", subcore_axis_name="s")
    run = pl.kernel(
        _tbody,
        out_type=jax.ShapeDtypeStruct((NPAD, 2, EMBEDDING_DIM), jnp.float32),
        mesh=mesh,
        scratch_types=[
            pltpu.VMEM((K1_NBUF, 8, 8, LANES), jnp.float32),
            pltpu.VMEM((K1_NBUF, LANES, SOUT_STRIDE), jnp.float32),
        ] + [pltpu.SemaphoreType.DMA] * (2 * K1_NBUF),
        compiler_params=pltpu.CompilerParams(
            use_tc_tiling_on_sc=False, needs_layout_passes=False
        ),
    )
    return run(t4)


def _body(idx_hbm, table_hbm, out_hbm, idx_v, rows_v, tr_v, *sems):
    wid = lax.axis_index("s") * NC + lax.axis_index("c")
    sem_g = sems[:NBUF]
    sem_s = sems[NBUF:]

    pltpu.sync_copy(idx_hbm.at[wid], idx_v)

    iota = lax.iota(jnp.int32, 16)

    def issue_gather(g, b):
        pltpu.async_copy(table_hbm.at[idx_v.at[g]], rows_v.at[b], sem_g[b])

    def wait_gather(b):
        pltpu.make_async_copy(
            table_hbm.at[pl.ds(0, LANES)], rows_v.at[b], sem_g[b]
        ).wait()

    dt_idx = [(c * 16 + iota) // 8 for c in range(EMBEDDING_DIM // 16)]
    dl_idx = [lax.rem(c * 16 + iota, 8) for c in range(EMBEDDING_DIM // 16)]

    def transpose(b):
        @plsc.parallel_loop(0, LANES, unroll=4)
        def _(bl):
            blv = jnp.broadcast_to(bl, (16,)).astype(jnp.int32)
            for c in range(EMBEDDING_DIM // 16):
                v = rows_v[b, bl, pl.ds(c * 16, 16)]
                plsc.store_scatter(tr_v.at[b], [dt_idx[c], dl_idx[c], blv], v)

    def issue_store(g, b):
        s = g // BT_PER_W
        bt = wid * BT_PER_W + lax.rem(g, BT_PER_W)
        pltpu.async_copy(
            tr_v.at[b, :, :, pl.ds(0, LANES)], out_hbm.at[s, :, bt], sem_s[b]
        )

    def wait_store(b):
        pltpu.make_async_copy(
            tr_v.at[b, :, :, pl.ds(0, LANES)], out_hbm.at[0, :, 0], sem_s[b]
        ).wait()

    for b in range(NBUF):
        issue_gather(b, b)

    def outer(o, carry):
        for b in range(NBUF):
            g = o * NBUF + b
            wait_gather(b)

            @pl.when(g >= NBUF)
            def _():
                wait_store(b)

            transpose(b)
            issue_store(g, b)

            @pl.when(g + NBUF < GROUPS)
            def _():
                issue_gather(g + NBUF, b)

        return carry

    lax.fori_loop(0, GROUPS // NBUF, outer, 0)
    for b in range(NBUF):
        wait_store(b)


@jax.jit
def _gather(idx_grouped, table2):
    mesh = plsc.VectorSubcoreMesh(core_axis_name="c", subcore_axis_name="s")
    run = pl.kernel(
        _body,
        out_type=jax.ShapeDtypeStruct(
            (SEQ_LEN, EMBEDDING_DIM // 8, N_BT, 8, LANES), jnp.float32
        ),
        mesh=mesh,
        scratch_types=[
            pltpu.VMEM((GROUPS, LANES), jnp.int32),
            pltpu.VMEM((NBUF, LANES, EMBEDDING_DIM), jnp.float32),
            pltpu.VMEM((NBUF, EMBEDDING_DIM // 8, 8, LANES + 1), jnp.float32),
        ] + [pltpu.SemaphoreType.DMA] * (2 * NBUF),
        compiler_params=pltpu.CompilerParams(
            use_tc_tiling_on_sc=False, needs_layout_passes=False
        ),
    )
    return run(idx_grouped, table2)


def kernel(token_ids, table):
    tt = token_ids.astype(jnp.int32).T
    idx = (
        tt.reshape(SEQ_LEN, NW, BT_PER_W, LANES)
        .transpose(1, 0, 2, 3)
        .reshape(NW, GROUPS, LANES)
    )
    tp = jnp.pad(table, ((0, NPAD - NUM_EMBEDDINGS), (0, 0)))
    t4 = tp.reshape(NT, LANES, 8, 8).transpose(2, 0, 3, 1)
    out1 = _transpose_table(t4)
    t2 = out1.reshape(2 * NPAD, EMBEDDING_DIM)
    out5 = _gather(idx * 2, t2)
    return out5.transpose(2, 4, 0, 1, 3).reshape(BATCH, SEQ_LEN, EMBEDDING_DIM)


# speedup vs baseline: 3.3610x; 1.0009x over previous
import jax
import jax.numpy as jnp
from jax import lax
from jax.experimental import pallas as pl
from jax.experimental.pallas import tpu as pltpu
from jax.experimental.pallas import tpu_sc as plsc

NUM_EMBEDDINGS = 1000000
EMBEDDING_DIM = 64
BATCH = 16384
SEQ_LEN = 50

_INFO = plsc.get_sparse_core_info()
NC, NS = _INFO.num_cores, _INFO.num_subcores
NW = NC * NS

LANES = 128
N_BT = BATCH // LANES
BT_PER_W = N_BT // NW
GROUPS = SEQ_LEN * BT_PER_W
NBUF = 4

NPAD = 1000064
NT = NPAD // LANES
K1_STEPS = -(-NT // NW)
K1_NBUF = 6
SOUT_STRIDE = 69


def _tbody(t4_hbm, out1_hbm, sin_v, sout_v, *sems):
    wid = lax.axis_index("s") * NC + lax.axis_index("c")
    sem_i = sems[:K1_NBUF]
    sem_o = sems[K1_NBUF:]
    iota = lax.iota(jnp.int32, 16)

    def issue_load(k, b):
        pltpu.async_copy(t4_hbm.at[:, k * NW + wid], sin_v.at[b], sem_i[b])

    def wait_load(b):
        pltpu.make_async_copy(t4_hbm.at[:, 0], sin_v.at[b], sem_i[b]).wait()

    def transpose(b):
        @plsc.parallel_loop(0, EMBEDDING_DIM, unroll=4)
        def _(d):
            dt = d // 8
            dl = lax.rem(d, 8)
            dcol = jnp.broadcast_to(d, (16,)).astype(jnp.int32)
            for c in range(8):
                v = sin_v[b, dt, dl, pl.ds(c * 16, 16)]
                plsc.store_scatter(sout_v.at[b], [c * 16 + iota, dcol], v)

    def issue_store(k, b):
        it = k * NW + wid
        pltpu.async_copy(
            sout_v.at[b, :, pl.ds(0, EMBEDDING_DIM)],
            out1_hbm.at[pl.ds(it * LANES, LANES)],
            sem_o[b],
        )

    def wait_store(b):
        pltpu.make_async_copy(
            sout_v.at[b, :, pl.ds(0, EMBEDDING_DIM)],
            out1_hbm.at[pl.ds(0, LANES)],
            sem_o[b],
        ).wait()

    def live(k):
        return k * NW + wid < NT

    for b in range(K1_NBUF):

        @pl.when(live(b))
        def _():
            issue_load(b, b)

    def outer(o, carry):
        for b in range(K1_NBUF):
            k = o * K1_NBUF + b

            @pl.when(live(k))
            def _():
                wait_load(b)

                @pl.when(k >= K1_NBUF)
                def _():
                    wait_store(b)

                transpose(b)
                issue_store(k, b)

            @pl.when(live(k + K1_NBUF))
            def _():
                issue_load(k + K1_NBUF, b)

        return carry

    lax.fori_loop(0, -(-K1_STEPS // K1_NBUF), outer, 0)
    for b in range(K1_NBUF):
        wait_store(b)


@jax.jit
def _transpose_table(t4):
    mesh = plsc.VectorSubcoreMesh(core_axis_name="c", subcore_axis_name="s")
    run = pl.kernel(
        _tbody,
        out_type=jax.ShapeDtypeStruct((NPAD, EMBEDDING_DIM), jnp.float32),
        mesh=mesh,
        scratch_types=[
            pltpu.VMEM((K1_NBUF, 8, 8, LANES), jnp.float32),
            pltpu.VMEM((K1_NBUF, LANES, SOUT_STRIDE), jnp.float32),
        ] + [pltpu.SemaphoreType.DMA] * (2 * K1_NBUF),
        compiler_params=pltpu.CompilerParams(
            use_tc_tiling_on_sc=False, needs_layout_passes=False
        ),
    )
    return run(t4)


def _body(idx_hbm, table_hbm, out_hbm, idx_v, rows_v, tr_v, *sems):
    wid = lax.axis_index("s") * NC + lax.axis_index("c")
    sem_g = sems[:NBUF]
    sem_s = sems[NBUF:]

    pltpu.sync_copy(idx_hbm.at[wid], idx_v)

    iota = lax.iota(jnp.int32, 16)

    def issue_gather(g, b):
        pltpu.async_copy(table_hbm.at[idx_v.at[g]], rows_v.at[b], sem_g[b])

    def wait_gather(b):
        pltpu.make_async_copy(
            table_hbm.at[pl.ds(0, LANES)], rows_v.at[b], sem_g[b]
        ).wait()

    dt_idx = [(c * 16 + iota) // 8 for c in range(EMBEDDING_DIM // 16)]
    dl_idx = [lax.rem(c * 16 + iota, 8) for c in range(EMBEDDING_DIM // 16)]

    def transpose(b):
        @plsc.parallel_loop(0, LANES, unroll=4)
        def _(bl):
            blv = jnp.broadcast_to(bl, (16,)).astype(jnp.int32)
            for c in range(EMBEDDING_DIM // 16):
                v = rows_v[b, bl, pl.ds(c * 16, 16)]
                plsc.store_scatter(tr_v.at[b], [dt_idx[c], dl_idx[c], blv], v)

    def issue_store(g, b):
        s = g // BT_PER_W
        bt = wid * BT_PER_W + lax.rem(g, BT_PER_W)
        pltpu.async_copy(
            tr_v.at[b, :, :, pl.ds(0, LANES)], out_hbm.at[s, :, bt], sem_s[b]
        )

    def wait_store(b):
        pltpu.make_async_copy(
            tr_v.at[b, :, :, pl.ds(0, LANES)], out_hbm.at[0, :, 0], sem_s[b]
        ).wait()

    for b in range(NBUF):
        issue_gather(b, b)

    def outer(o, carry):
        for b in range(NBUF):
            g = o * NBUF + b
            wait_gather(b)

            @pl.when(g >= NBUF)
            def _():
                wait_store(b)

            transpose(b)
            issue_store(g, b)

            @pl.when(g + NBUF < GROUPS)
            def _():
                issue_gather(g + NBUF, b)

        return carry

    lax.fori_loop(0, GROUPS // NBUF, outer, 0)
    for b in range(NBUF):
        wait_store(b)


@jax.jit
def _gather(idx_grouped, table2):
    mesh = plsc.VectorSubcoreMesh(core_axis_name="c", subcore_axis_name="s")
    run = pl.kernel(
        _body,
        out_type=jax.ShapeDtypeStruct(
            (SEQ_LEN, EMBEDDING_DIM // 8, N_BT, 8, LANES), jnp.float32
        ),
        mesh=mesh,
        scratch_types=[
            pltpu.VMEM((GROUPS, LANES), jnp.int32),
            pltpu.VMEM((NBUF, LANES, EMBEDDING_DIM), jnp.float32),
            pltpu.VMEM((NBUF, EMBEDDING_DIM // 8, 8, LANES + 1), jnp.float32),
        ] + [pltpu.SemaphoreType.DMA] * (2 * NBUF),
        compiler_params=pltpu.CompilerParams(
            use_tc_tiling_on_sc=False, needs_layout_passes=False
        ),
    )
    return run(idx_grouped, table2)


def kernel(token_ids, table):
    tt = token_ids.astype(jnp.int32).T
    idx = (
        tt.reshape(SEQ_LEN, NW, BT_PER_W, LANES)
        .transpose(1, 0, 2, 3)
        .reshape(NW, GROUPS, LANES)
    )
    tp = jnp.pad(table, ((0, NPAD - NUM_EMBEDDINGS), (0, 0)))
    t4 = tp.reshape(NT, LANES, 8, 8).transpose(2, 0, 3, 1)
    out1 = _transpose_table(t4)
    out5 = _gather(idx, out1)
    return out5.transpose(2, 4, 0, 1, 3).reshape(BATCH, SEQ_LEN, EMBEDDING_DIM)
